# baseline (speedup 1.0000x reference)
import functools

import jax
import jax.numpy as jnp
from jax import lax
from jax.experimental import pallas as pl
from jax.experimental.pallas import tpu as pltpu
from jax.experimental.pallas import tpu_sc as plsc

HIDDEN = 768
HALF = HIDDEN // 2
EPS = 1e-12
GATHER_WINDOW = 128


def _sc_gather(table, ids2d):
    n2 = ids2d.shape[1]
    table_half = table.reshape(table.shape[0] * 2, HALF)
    mesh = plsc.VectorSubcoreMesh(core_axis_name="core", subcore_axis_name="subcore")

    @functools.partial(
        pl.kernel,
        out_type=jax.ShapeDtypeStruct((n2, HALF), table.dtype),
        mesh=mesh,
    )
    def gather_kernel(table_hbm, ids_hbm, out_hbm):
        def body(ids_vmem, out_vmem):
            pltpu.sync_copy(table_hbm.at[ids_vmem.at[0]], out_vmem)

        pltpu.emit_pipeline(
            body,
            grid=(n2 // GATHER_WINDOW,),
            in_specs=[pl.BlockSpec((1, GATHER_WINDOW), lambda i: (0, i))],
            out_specs=[pl.BlockSpec((GATHER_WINDOW, HALF), lambda i: (i, 0))],
            core_axis_name=("core", "subcore"),
            dimension_semantics=(pltpu.PARALLEL,),
        )(ids_hbm, out_hbm)

    return gather_kernel(table_half, ids2d)


def _tc_add_ln_body(x_ref, tt_ref, pos_ref, type_ref, gamma_ref, beta_ref, o_ref):
    x = x_ref[...]
    tt = tt_ref[:, :1]
    pos = pos_ref[...]
    t0 = type_ref[0, :]
    t1 = type_ref[1, :]
    te = tt * (t1 - t0)[None, :] + t0[None, :]
    e = x + pos + te
    mean = jnp.mean(e, axis=-1, keepdims=True)
    c = e - mean
    var = jnp.mean(c * c, axis=-1, keepdims=True)
    normed = c * lax.rsqrt(var + EPS)
    o_ref[...] = normed * gamma_ref[0, :] + beta_ref[0, :]


def _tc_add_ln(gathered, tt3, position_embeddings, token_type_embeddings,
               ln_gamma, ln_beta, batch, seq):
    return pl.pallas_call(
        _tc_add_ln_body,
        grid=(batch,),
        in_specs=[
            pl.BlockSpec((seq, HIDDEN), lambda i: (i, 0)),
            pl.BlockSpec((seq, 8), lambda i: (i, 0)),
            pl.BlockSpec((seq, HIDDEN), lambda i: (0, 0)),
            pl.BlockSpec((2, HIDDEN), lambda i: (0, 0)),
            pl.BlockSpec((1, HIDDEN), lambda i: (0, 0)),
            pl.BlockSpec((1, HIDDEN), lambda i: (0, 0)),
        ],
        out_specs=pl.BlockSpec((seq, HIDDEN), lambda i: (i, 0)),
        out_shape=jax.ShapeDtypeStruct((batch * seq, HIDDEN), jnp.float32),
    )(gathered, tt3, position_embeddings, token_type_embeddings,
      ln_gamma.reshape(1, HIDDEN), ln_beta.reshape(1, HIDDEN))


def kernel(input_ids, token_type_ids, position_ids, word_embeddings,
           position_embeddings, token_type_embeddings, ln_gamma, ln_beta):
    batch, seq = input_ids.shape
    n = batch * seq
    ids = input_ids.astype(jnp.int32).reshape(n)
    half_ids = jnp.stack([ids * 2, ids * 2 + 1], axis=-1).reshape(1, 2 * n)
    gathered = _sc_gather(word_embeddings, half_ids).reshape(n, HIDDEN)
    tt8 = jnp.broadcast_to(
        token_type_ids.astype(jnp.float32).reshape(n, 1), (n, 8))
    out = _tc_add_ln(gathered, tt8, position_embeddings, token_type_embeddings,
                     ln_gamma, ln_beta, batch, seq)
    return out.reshape(batch, seq, HIDDEN)


# speedup vs baseline: 1.0042x; 1.0042x over previous
import functools

import jax
import jax.numpy as jnp
from jax import lax
from jax.experimental import pallas as pl
from jax.experimental.pallas import tpu as pltpu
from jax.experimental.pallas import tpu_sc as plsc

HIDDEN = 768
HALF = HIDDEN // 2
EPS = 1e-12
GATHER_WINDOW = 128


def _sc_gather(table, ids2d):
    n2 = ids2d.shape[1]
    table_half = table.reshape(table.shape[0] * 2, HALF)
    mesh = plsc.VectorSubcoreMesh(core_axis_name="core", subcore_axis_name="subcore")

    @functools.partial(
        pl.kernel,
        out_type=jax.ShapeDtypeStruct((n2, HALF), table.dtype),
        mesh=mesh,
    )
    def gather_kernel(table_hbm, ids_hbm, out_hbm):
        def body(ids_vmem, out_vmem):
            pltpu.sync_copy(table_hbm.at[ids_vmem.at[0]], out_vmem)

        pltpu.emit_pipeline(
            body,
            grid=(n2 // GATHER_WINDOW,),
            in_specs=[pl.BlockSpec((1, GATHER_WINDOW), lambda i: (0, i))],
            out_specs=[pl.BlockSpec((GATHER_WINDOW, HALF), lambda i: (i, 0))],
            core_axis_name=("core", "subcore"),
            dimension_semantics=(pltpu.PARALLEL,),
        )(ids_hbm, out_hbm)

    return gather_kernel(table_half, ids2d)


def _tc_add_ln_body(x_ref, tt_ref, pos_ref, type_ref, gamma_ref, beta_ref, o_ref):
    x = x_ref[...]
    tt = tt_ref[:, :1]
    pos = pos_ref[...]
    t0 = type_ref[0, :]
    t1 = type_ref[1, :]
    te = tt * (t1 - t0)[None, :] + t0[None, :]
    e = x + pos + te
    mean = jnp.mean(e, axis=-1, keepdims=True)
    c = e - mean
    var = jnp.mean(c * c, axis=-1, keepdims=True)
    normed = c * lax.rsqrt(var + EPS)
    o_ref[...] = normed * gamma_ref[0, :] + beta_ref[0, :]


def _tc_add_ln(gathered, tt3, position_embeddings, token_type_embeddings,
               ln_gamma, ln_beta, batch, seq):
    return pl.pallas_call(
        _tc_add_ln_body,
        grid=(batch,),
        in_specs=[
            pl.BlockSpec((seq, HIDDEN), lambda i: (i, 0)),
            pl.BlockSpec((seq, 8), lambda i: (i, 0)),
            pl.BlockSpec((seq, HIDDEN), lambda i: (0, 0)),
            pl.BlockSpec((2, HIDDEN), lambda i: (0, 0)),
            pl.BlockSpec((1, HIDDEN), lambda i: (0, 0)),
            pl.BlockSpec((1, HIDDEN), lambda i: (0, 0)),
        ],
        out_specs=pl.BlockSpec((seq, HIDDEN), lambda i: (i, 0)),
        out_shape=jax.ShapeDtypeStruct((batch * seq, HIDDEN), jnp.float32),
        compiler_params=pltpu.CompilerParams(
            dimension_semantics=("parallel",)),
    )(gathered, tt3, position_embeddings, token_type_embeddings,
      ln_gamma.reshape(1, HIDDEN), ln_beta.reshape(1, HIDDEN))


def kernel(input_ids, token_type_ids, position_ids, word_embeddings,
           position_embeddings, token_type_embeddings, ln_gamma, ln_beta):
    batch, seq = input_ids.shape
    n = batch * seq
    ids = input_ids.astype(jnp.int32).reshape(n)
    half_ids = jnp.stack([ids * 2, ids * 2 + 1], axis=-1).reshape(1, 2 * n)
    gathered = _sc_gather(word_embeddings, half_ids).reshape(n, HIDDEN)
    tt8 = jnp.broadcast_to(
        token_type_ids.astype(jnp.float32).reshape(n, 1), (n, 8))
    out = _tc_add_ln(gathered, tt8, position_embeddings, token_type_embeddings,
                     ln_gamma, ln_beta, batch, seq)
    return out.reshape(batch, seq, HIDDEN)
